# baseline (speedup 1.0000x reference)
import functools

import jax
import jax.numpy as jnp
from jax import lax
from jax.experimental import pallas as pl
from jax.experimental.pallas import tpu as pltpu
from jax.experimental.pallas import tpu_sc as plsc

B = 8
L = 512
H = 16
NUM_EMB = 514
SUPER = NUM_EMB - 1
TS = 520
R = 2
IPB = L // R
ITEM = R * L
N_ITEMS = B * IPB
CHUNKS = ITEM // 16
CPR = L // 16


def _sc_lookup(edge_flat, wt_flat, n_workers):
    per_w = N_ITEMS // n_workers
    n_sub = 16

    @functools.partial(
        pl.kernel,
        mesh=plsc.VectorSubcoreMesh(core_axis_name="c", subcore_axis_name="s"),
        compiler_params=pltpu.CompilerParams(needs_layout_passes=False),
        out_type=jax.ShapeDtypeStruct((B * H, L * L), jnp.float32),
        scratch_types=[
            pltpu.VMEM((H * TS,), jnp.float32),
            pltpu.VMEM((ITEM,), jnp.int32),
            pltpu.VMEM((ITEM,), jnp.int32),
            pltpu.VMEM((H, ITEM), jnp.float32),
            pltpu.VMEM((H, ITEM), jnp.float32),
            pltpu.VMEM_SHARED((n_sub, 2, H, ITEM), jnp.float32),
            pltpu.SemaphoreType.DMA,
            pltpu.SemaphoreType.DMA,
            pltpu.SemaphoreType.DMA,
            pltpu.SemaphoreType.DMA,
        ],
    )
    def k(edge_hbm, wt_hbm, out_hbm, wt_v, ib0, ib1, ob0, ob1, sh,
          si0, si1, so0, so1):
        nc = 2
        sid = lax.axis_index("s")
        wid = sid * nc + lax.axis_index("c")
        base_item = wid * per_w
        lane0 = lax.iota(jnp.int32, 16) == 0
        super_vec = jnp.full((16,), SUPER, jnp.int32)

        def in_copy(it, buf, sem):
            item = base_item + it
            return pltpu.make_async_copy(
                edge_hbm.at[pl.ds(item * ITEM, ITEM)], buf, sem)

        def out_copy(it, p, sem):
            item = base_item + it
            b = item // IPB
            blk = item % IPB
            return pltpu.make_async_copy(
                sh.at[sid, p],
                out_hbm.at[pl.ds(b * H, H), pl.ds(blk * ITEM, ITEM)],
                sem)

        def compute(it, ibuf, obuf):
            item = base_item + it

            for r in range(R):
                v = ibuf[pl.ds(r * L, 16)]
                ibuf[pl.ds(r * L, 16)] = jnp.where(lane0, SUPER, v)

            @pl.when((item % IPB) == 0)
            def _():
                @plsc.parallel_loop(0, CPR, unroll=4)
                def _(c):
                    ibuf[pl.ds(c * 16, 16)] = super_vec

            @plsc.parallel_loop(0, CHUNKS, unroll=4)
            def _(c):
                pos = ibuf[pl.ds(c * 16, 16)]
                for h in range(H):
                    obuf[h, pl.ds(c * 16, 16)] = plsc.load_gather(
                        wt_v, [pos + h * TS])

        def step(it, ibuf, obuf, p, si, so, first=False, prefetch=True):
            in_copy(it, ibuf, si).wait()
            if not first:
                out_copy(it - 2, p, so).wait()
            compute(it, ibuf, obuf)
            pltpu.sync_copy(obuf, sh.at[sid, p])
            out_copy(it, p, so).start()
            if prefetch:
                in_copy(it + 2, ibuf, si).start()

        pltpu.sync_copy(wt_hbm, wt_v)
        in_copy(0, ib0, si0).start()
        in_copy(1, ib1, si1).start()

        step(0, ib0, ob0, 0, si0, so0, first=True)
        step(1, ib1, ob1, 1, si1, so1, first=True)

        def loop_body(t, carry):
            step(2 * t, ib0, ob0, 0, si0, so0)
            step(2 * t + 1, ib1, ob1, 1, si1, so1)
            return carry

        lax.fori_loop(1, per_w // 2 - 1, loop_body, 0)

        step(per_w - 2, ib0, ob0, 0, si0, so0, prefetch=False)
        step(per_w - 1, ib1, ob1, 1, si1, so1, prefetch=False)

        out_copy(per_w - 2, 0, so0).wait()
        out_copy(per_w - 1, 1, so1).wait()

    return k(edge_flat, wt_flat)


def kernel(edge_dist, W):
    info = plsc.get_sparse_core_info()
    n_workers = info.num_cores * info.num_subcores
    wt = jnp.zeros((H, TS), jnp.float32).at[:, :NUM_EMB].set(W.T)
    out2 = _sc_lookup(edge_dist.reshape(-1), wt.reshape(-1), n_workers)
    return out2.reshape(B, H, L, L)


# speedup vs baseline: 1.2155x; 1.2155x over previous
import functools

import jax
import jax.numpy as jnp
from jax import lax
from jax.experimental import pallas as pl
from jax.experimental.pallas import tpu as pltpu
from jax.experimental.pallas import tpu_sc as plsc

B = 8
L = 512
H = 16
NUM_EMB = 514
SUPER = NUM_EMB - 1
TS = 520
R = 4
IPB = L // R
ITEM = R * L
N_ITEMS = B * IPB
CHUNKS = ITEM // 16
CPR = L // 16


def _sc_lookup(edge_flat, wt_flat, n_workers):
    per_w = N_ITEMS // n_workers

    @functools.partial(
        pl.kernel,
        mesh=plsc.VectorSubcoreMesh(core_axis_name="c", subcore_axis_name="s"),
        compiler_params=pltpu.CompilerParams(needs_layout_passes=False),
        out_type=jax.ShapeDtypeStruct((B * H, L * L), jnp.float32),
        scratch_types=[
            pltpu.VMEM((H * TS,), jnp.float32),
            pltpu.VMEM((ITEM,), jnp.int32),
            pltpu.VMEM((ITEM,), jnp.int32),
            pltpu.VMEM((H, ITEM), jnp.float32),
            pltpu.VMEM((H, ITEM), jnp.float32),
            pltpu.SemaphoreType.DMA,
            pltpu.SemaphoreType.DMA,
            pltpu.SemaphoreType.DMA,
            pltpu.SemaphoreType.DMA,
            pltpu.SemaphoreType.DMA,
            pltpu.SemaphoreType.DMA,
        ],
    )
    def k(edge_hbm, wt_hbm, out_hbm, wt_v, ib0, ib1, ob0, ob1,
          si0, si1, so0a, so0b, so1a, so1b):
        nc = 2
        wid = lax.axis_index("s") * nc + lax.axis_index("c")
        base_item = wid * per_w
        lane0 = lax.iota(jnp.int32, 16) == 0
        super_vec = jnp.full((16,), SUPER, jnp.int32)

        def in_copy(it, buf, sem):
            item = base_item + it
            return pltpu.make_async_copy(
                edge_hbm.at[pl.ds(item * ITEM, ITEM)], buf, sem)

        def out_copies(it, buf, sema, semb):
            item = base_item + it
            b = item // IPB
            blk = item % IPB
            hh = H // 2
            return (
                pltpu.make_async_copy(
                    buf.at[pl.ds(0, hh)],
                    out_hbm.at[pl.ds(b * H, hh), pl.ds(blk * ITEM, ITEM)],
                    sema),
                pltpu.make_async_copy(
                    buf.at[pl.ds(hh, hh)],
                    out_hbm.at[pl.ds(b * H + hh, hh), pl.ds(blk * ITEM, ITEM)],
                    semb),
            )

        def compute(it, ibuf, obuf):
            item = base_item + it

            for r in range(R):
                v = ibuf[pl.ds(r * L, 16)]
                ibuf[pl.ds(r * L, 16)] = jnp.where(lane0, SUPER, v)

            @pl.when((item % IPB) == 0)
            def _():
                @plsc.parallel_loop(0, CPR, unroll=4)
                def _(c):
                    ibuf[pl.ds(c * 16, 16)] = super_vec

            @plsc.parallel_loop(0, CHUNKS, unroll=4)
            def _(c):
                pos = ibuf[pl.ds(c * 16, 16)]
                for h in range(H):
                    obuf[h, pl.ds(c * 16, 16)] = plsc.load_gather(
                        wt_v, [pos + h * TS])

        def step(it, ibuf, obuf, sia, soa, sob, first=False, prefetch=True):
            in_copy(it, ibuf, sia).wait()
            if not first:
                ca, cb = out_copies(it - 2, obuf, soa, sob)
                ca.wait()
                cb.wait()
            compute(it, ibuf, obuf)
            ca, cb = out_copies(it, obuf, soa, sob)
            ca.start()
            cb.start()
            if prefetch:
                in_copy(it + 2, ibuf, sia).start()

        pltpu.sync_copy(wt_hbm, wt_v)
        in_copy(0, ib0, si0).start()
        in_copy(1, ib1, si1).start()

        step(0, ib0, ob0, si0, so0a, so0b, first=True)
        step(1, ib1, ob1, si1, so1a, so1b, first=True)

        def loop_body(t, carry):
            step(2 * t, ib0, ob0, si0, so0a, so0b)
            step(2 * t + 1, ib1, ob1, si1, so1a, so1b)
            return carry

        lax.fori_loop(1, per_w // 2 - 1, loop_body, 0)

        step(per_w - 2, ib0, ob0, si0, so0a, so0b, prefetch=False)
        step(per_w - 1, ib1, ob1, si1, so1a, so1b, prefetch=False)

        for it, obuf, soa, sob in ((per_w - 2, ob0, so0a, so0b),
                                   (per_w - 1, ob1, so1a, so1b)):
            ca, cb = out_copies(it, obuf, soa, sob)
            ca.wait()
            cb.wait()

    return k(edge_flat, wt_flat)


def kernel(edge_dist, W):
    info = plsc.get_sparse_core_info()
    n_workers = info.num_cores * info.num_subcores
    wt = jnp.zeros((H, TS), jnp.float32).at[:, :NUM_EMB].set(W.T)
    out2 = _sc_lookup(edge_dist.reshape(-1), wt.reshape(-1), n_workers)
    return out2.reshape(B, H, L, L)


# speedup vs baseline: 1.2158x; 1.0002x over previous
import functools

import jax
import jax.numpy as jnp
from jax import lax
from jax.experimental import pallas as pl
from jax.experimental.pallas import tpu as pltpu
from jax.experimental.pallas import tpu_sc as plsc

B = 8
L = 512
H = 16
NUM_EMB = 514
SUPER = NUM_EMB - 1
TS = 520
R = 4
IPB = L // R
ITEM = R * L
N_ITEMS = B * IPB
CHUNKS = ITEM // 16
CPR = L // 16


def _sc_lookup(edge_flat, wt_flat, n_workers):
    per_w = N_ITEMS // n_workers

    @functools.partial(
        pl.kernel,
        mesh=plsc.VectorSubcoreMesh(core_axis_name="c", subcore_axis_name="s"),
        compiler_params=pltpu.CompilerParams(
            needs_layout_passes=False, use_tc_tiling_on_sc=True),
        out_type=jax.ShapeDtypeStruct((B * H, L * L), jnp.float32),
        scratch_types=[
            pltpu.VMEM((H * TS,), jnp.float32),
            pltpu.VMEM((ITEM,), jnp.int32),
            pltpu.VMEM((ITEM,), jnp.int32),
            pltpu.VMEM((H, ITEM), jnp.float32),
            pltpu.VMEM((H, ITEM), jnp.float32),
            pltpu.SemaphoreType.DMA,
            pltpu.SemaphoreType.DMA,
            pltpu.SemaphoreType.DMA,
            pltpu.SemaphoreType.DMA,
            pltpu.SemaphoreType.DMA,
            pltpu.SemaphoreType.DMA,
        ],
    )
    def k(edge_hbm, wt_hbm, out_hbm, wt_v, ib0, ib1, ob0, ob1,
          si0, si1, so0a, so0b, so1a, so1b):
        nc = 2
        wid = lax.axis_index("s") * nc + lax.axis_index("c")
        base_item = wid * per_w
        lane0 = lax.iota(jnp.int32, 16) == 0
        super_vec = jnp.full((16,), SUPER, jnp.int32)

        def in_copy(it, buf, sem):
            item = base_item + it
            return pltpu.make_async_copy(
                edge_hbm.at[pl.ds(item * ITEM, ITEM)], buf, sem)

        def out_copies(it, buf, sema, semb):
            item = base_item + it
            b = item // IPB
            blk = item % IPB
            hh = H // 2
            return (
                pltpu.make_async_copy(
                    buf.at[pl.ds(0, hh)],
                    out_hbm.at[pl.ds(b * H, hh), pl.ds(blk * ITEM, ITEM)],
                    sema),
                pltpu.make_async_copy(
                    buf.at[pl.ds(hh, hh)],
                    out_hbm.at[pl.ds(b * H + hh, hh), pl.ds(blk * ITEM, ITEM)],
                    semb),
            )

        def compute(it, ibuf, obuf):
            item = base_item + it

            for r in range(R):
                v = ibuf[pl.ds(r * L, 16)]
                ibuf[pl.ds(r * L, 16)] = jnp.where(lane0, SUPER, v)

            @pl.when((item % IPB) == 0)
            def _():
                @plsc.parallel_loop(0, CPR, unroll=4)
                def _(c):
                    ibuf[pl.ds(c * 16, 16)] = super_vec

            @plsc.parallel_loop(0, CHUNKS, unroll=4)
            def _(c):
                pos = ibuf[pl.ds(c * 16, 16)]
                for h in range(H):
                    obuf[h, pl.ds(c * 16, 16)] = plsc.load_gather(
                        wt_v, [pos + h * TS])

        def step(it, ibuf, obuf, sia, soa, sob, first=False, prefetch=True):
            in_copy(it, ibuf, sia).wait()
            if not first:
                ca, cb = out_copies(it - 2, obuf, soa, sob)
                ca.wait()
                cb.wait()
            compute(it, ibuf, obuf)
            ca, cb = out_copies(it, obuf, soa, sob)
            ca.start()
            cb.start()
            if prefetch:
                in_copy(it + 2, ibuf, sia).start()

        pltpu.sync_copy(wt_hbm, wt_v)
        in_copy(0, ib0, si0).start()
        in_copy(1, ib1, si1).start()

        step(0, ib0, ob0, si0, so0a, so0b, first=True)
        step(1, ib1, ob1, si1, so1a, so1b, first=True)

        def loop_body(t, carry):
            step(2 * t, ib0, ob0, si0, so0a, so0b)
            step(2 * t + 1, ib1, ob1, si1, so1a, so1b)
            return carry

        lax.fori_loop(1, per_w // 2 - 1, loop_body, 0)

        step(per_w - 2, ib0, ob0, si0, so0a, so0b, prefetch=False)
        step(per_w - 1, ib1, ob1, si1, so1a, so1b, prefetch=False)

        for it, obuf, soa, sob in ((per_w - 2, ob0, so0a, so0b),
                                   (per_w - 1, ob1, so1a, so1b)):
            ca, cb = out_copies(it, obuf, soa, sob)
            ca.wait()
            cb.wait()

    return k(edge_flat, wt_flat)


def kernel(edge_dist, W):
    info = plsc.get_sparse_core_info()
    n_workers = info.num_cores * info.num_subcores
    wt = jnp.zeros((H, TS), jnp.float32).at[:, :NUM_EMB].set(W.T)
    out2 = _sc_lookup(edge_dist.reshape(-1), wt.reshape(-1), n_workers)
    return out2.reshape(B, H, L, L)


# speedup vs baseline: 1.2283x; 1.0103x over previous
import functools

import jax
import jax.numpy as jnp
from jax import lax
from jax.experimental import pallas as pl
from jax.experimental.pallas import tpu as pltpu
from jax.experimental.pallas import tpu_sc as plsc

B = 8
L = 512
H = 16
NUM_EMB = 514
SUPER = NUM_EMB - 1
TS = 520
R = 4
IPB = L // R
ITEM = R * L
N_ITEMS = B * IPB
CHUNKS = ITEM // 16
CPR = L // 16


def _sc_lookup(edge_flat, wt_flat, n_workers):
    per_w = N_ITEMS // n_workers

    @functools.partial(
        pl.kernel,
        mesh=plsc.VectorSubcoreMesh(core_axis_name="c", subcore_axis_name="s"),
        compiler_params=pltpu.CompilerParams(needs_layout_passes=False),
        out_type=jax.ShapeDtypeStruct((B * H, L * L), jnp.float32),
        scratch_types=[
            pltpu.VMEM((H * TS,), jnp.float32),
            pltpu.VMEM((ITEM,), jnp.int32),
            pltpu.VMEM((ITEM,), jnp.int32),
            pltpu.VMEM((H, ITEM), jnp.float32),
            pltpu.VMEM((H, ITEM), jnp.float32),
            pltpu.SemaphoreType.DMA,
            pltpu.SemaphoreType.DMA,
            pltpu.SemaphoreType.DMA,
            pltpu.SemaphoreType.DMA,
        ],
    )
    def k(edge_hbm, wt_hbm, out_hbm, wt_v, ib0, ib1, ob0, ob1,
          si0, si1, so0, so1):
        nc = 2
        wid = lax.axis_index("s") * nc + lax.axis_index("c")
        base_item = wid * per_w
        lane0 = lax.iota(jnp.int32, 16) == 0
        super_vec = jnp.full((16,), SUPER, jnp.int32)

        def in_copy(it, buf, sem):
            item = base_item + it
            return pltpu.make_async_copy(
                edge_hbm.at[pl.ds(item * ITEM, ITEM)], buf, sem)

        def out_copy(it, buf, sem):
            item = base_item + it
            b = item // IPB
            blk = item % IPB
            return pltpu.make_async_copy(
                buf,
                out_hbm.at[pl.ds(b * H, H), pl.ds(blk * ITEM, ITEM)],
                sem)

        def compute(it, ibuf, obuf):
            item = base_item + it

            for r in range(R):
                v = ibuf[pl.ds(r * L, 16)]
                ibuf[pl.ds(r * L, 16)] = jnp.where(lane0, SUPER, v)

            @pl.when((item % IPB) == 0)
            def _():
                @plsc.parallel_loop(0, CPR, unroll=4)
                def _(c):
                    ibuf[pl.ds(c * 16, 16)] = super_vec

            @plsc.parallel_loop(0, CHUNKS, unroll=4)
            def _(c):
                pos = ibuf[pl.ds(c * 16, 16)]
                for h in range(H):
                    obuf[h, pl.ds(c * 16, 16)] = plsc.load_gather(
                        wt_v, [pos + h * TS])

        def step(it, ibuf, obuf, si, so, first=False, prefetch=True):
            in_copy(it, ibuf, si).wait()
            if not first:
                out_copy(it - 2, obuf, so).wait()
            compute(it, ibuf, obuf)
            out_copy(it, obuf, so).start()
            if prefetch:
                in_copy(it + 2, ibuf, si).start()

        pltpu.sync_copy(wt_hbm, wt_v)
        in_copy(0, ib0, si0).start()
        in_copy(1, ib1, si1).start()

        step(0, ib0, ob0, si0, so0, first=True)
        step(1, ib1, ob1, si1, so1, first=True)

        def loop_body(t, carry):
            step(2 * t, ib0, ob0, si0, so0)
            step(2 * t + 1, ib1, ob1, si1, so1)
            return carry

        lax.fori_loop(1, per_w // 2 - 1, loop_body, 0)

        step(per_w - 2, ib0, ob0, si0, so0, prefetch=False)
        step(per_w - 1, ib1, ob1, si1, so1, prefetch=False)

        out_copy(per_w - 2, ob0, so0).wait()
        out_copy(per_w - 1, ob1, so1).wait()

    return k(edge_flat, wt_flat)


def kernel(edge_dist, W):
    info = plsc.get_sparse_core_info()
    n_workers = info.num_cores * info.num_subcores
    wt = jnp.zeros((H, TS), jnp.float32).at[:, :NUM_EMB].set(W.T)
    out2 = _sc_lookup(edge_dist.reshape(-1), wt.reshape(-1), n_workers)
    return out2.reshape(B, H, L, L)
